# baseline (device time: 89084 ns/iter reference)
import jax
import jax.numpy as jnp
from jax import lax
from jax.experimental import pallas as pl
from jax.experimental.pallas import tpu as pltpu

T = 2048
T_HALF = T // 2
D = 1024
V_SHARD = 16384


def kernel(ids, E):
    my_x = lax.axis_index("x")
    my_y = lax.axis_index("y")

    ids_half = lax.dynamic_slice(ids, (my_x * T_HALF,), (T_HALF,))
    local = ids_half - my_y * V_SHARD
    ok = (local >= 0) & (local < V_SHARD)
    rows = jnp.where(ok, local, 0)
    partial = jnp.where(ok[:, None], E[rows, :], 0.0).astype(jnp.bfloat16)

    def body(p_ref, out_ref, recv_y, r_ref, recv_x, sems):
        mx = lax.axis_index("x")
        my = lax.axis_index("y")

        barrier = pltpu.get_barrier_semaphore()
        pl.semaphore_signal(barrier, inc=1, device_id=(mx, 1 - my),
                            device_id_type=pl.DeviceIdType.MESH)
        pl.semaphore_signal(barrier, inc=1, device_id=(1 - mx, my),
                            device_id_type=pl.DeviceIdType.MESH)
        pl.semaphore_wait(barrier, 2)

        rdma_y = pltpu.make_async_remote_copy(
            src_ref=p_ref,
            dst_ref=recv_y,
            send_sem=sems.at[0],
            recv_sem=sems.at[1],
            device_id=(mx, 1 - my),
            device_id_type=pl.DeviceIdType.MESH,
        )
        rdma_y.start()
        rdma_y.wait()

        r_ref[...] = p_ref[...] + recv_y[...]

        rdma_x = pltpu.make_async_remote_copy(
            src_ref=r_ref,
            dst_ref=recv_x,
            send_sem=sems.at[2],
            recv_sem=sems.at[3],
            device_id=(1 - mx, my),
            device_id_type=pl.DeviceIdType.MESH,
        )
        rdma_x.start()
        rdma_x.wait()

        out_ref[pl.ds(mx * T_HALF, T_HALF), :] = r_ref[...].astype(jnp.float32)
        out_ref[pl.ds((1 - mx) * T_HALF, T_HALF), :] = (
            recv_x[...].astype(jnp.float32))

    return pl.pallas_call(
        body,
        out_shape=jax.ShapeDtypeStruct((T, D), jnp.float32),
        in_specs=[pl.BlockSpec(memory_space=pltpu.VMEM)],
        out_specs=pl.BlockSpec(memory_space=pltpu.VMEM),
        scratch_shapes=[
            pltpu.VMEM((T_HALF, D), jnp.bfloat16),
            pltpu.VMEM((T_HALF, D), jnp.bfloat16),
            pltpu.VMEM((T_HALF, D), jnp.bfloat16),
            pltpu.SemaphoreType.DMA((4,)),
        ],
        compiler_params=pltpu.CompilerParams(collective_id=0),
    )(partial)


# device time: 27974 ns/iter; 3.1845x vs baseline; 3.1845x over previous
import jax
import jax.numpy as jnp
from jax import lax
from jax.experimental import pallas as pl
from jax.experimental.pallas import tpu as pltpu

T = 2048
T_HALF = T // 2
D = 1024
V_SHARD = 16384

K = 8
C = T_HALF // K


def kernel(ids, E):
    my_x = lax.axis_index("x")
    my_y = lax.axis_index("y")

    ids_half = lax.dynamic_slice(ids, (my_x * T_HALF,), (T_HALF,))
    local = ids_half - my_y * V_SHARD
    ok = (local >= 0) & (local < V_SHARD)
    rows = jnp.where(ok, local, 0).astype(jnp.int32)
    okf = ok.astype(jnp.float32).reshape(T_HALF, 1)

    def body(rows_ref, okf_ref, e_ref, out_ref,
             g_ref, p_ref, recv_y, r_ref, recv_x,
             gsem, ysend, yrecv, xsend, xrecv):
        mx = lax.axis_index("x")
        my = lax.axis_index("y")

        barrier = pltpu.get_barrier_semaphore()
        pl.semaphore_signal(barrier, inc=1, device_id=(mx, 1 - my),
                            device_id_type=pl.DeviceIdType.MESH)
        pl.semaphore_signal(barrier, inc=1, device_id=(1 - mx, my),
                            device_id_type=pl.DeviceIdType.MESH)
        pl.semaphore_wait(barrier, 2)

        def chunk(ref, c):
            return ref.at[pl.ds(c * C, C), :]

        def issue_gather(c, sem):
            def one(i, carry):
                t = c * C + i
                row = rows_ref[t]
                pltpu.make_async_copy(
                    e_ref.at[pl.ds(row, 1), :],
                    g_ref.at[pl.ds(t, 1), :],
                    sem,
                ).start()
                return carry
            lax.fori_loop(0, C, one, 0, unroll=8)

        issue_gather(0, gsem.at[0])
        issue_gather(1, gsem.at[1])

        rdma_y = []
        for c in range(K):
            pltpu.make_async_copy(
                e_ref.at[pl.ds(0, C), :], chunk(g_ref, c), gsem.at[c % 2]
            ).wait()
            p_ref[pl.ds(c * C, C), :] = (
                g_ref[pl.ds(c * C, C), :] * okf_ref[pl.ds(c * C, C), :]
            ).astype(jnp.bfloat16)

            ry = pltpu.make_async_remote_copy(
                src_ref=chunk(p_ref, c),
                dst_ref=chunk(recv_y, c),
                send_sem=ysend.at[c],
                recv_sem=yrecv.at[c],
                device_id=(mx, 1 - my),
                device_id_type=pl.DeviceIdType.MESH,
            )
            ry.start()
            rdma_y.append(ry)

            if c + 2 < K:
                issue_gather(c + 2, gsem.at[c % 2])

        rdma_x = []
        for c in range(K):
            rdma_y[c].wait_recv()
            r_ref[pl.ds(c * C, C), :] = (
                p_ref[pl.ds(c * C, C), :] + recv_y[pl.ds(c * C, C), :])
            rx = pltpu.make_async_remote_copy(
                src_ref=chunk(r_ref, c),
                dst_ref=chunk(recv_x, c),
                send_sem=xsend.at[c],
                recv_sem=xrecv.at[c],
                device_id=(1 - mx, my),
                device_id_type=pl.DeviceIdType.MESH,
            )
            rx.start()
            rdma_x.append(rx)
            out_ref[pl.ds(mx * T_HALF + c * C, C), :] = (
                r_ref[pl.ds(c * C, C), :].astype(jnp.float32))

        for c in range(K):
            rdma_x[c].wait_recv()
            out_ref[pl.ds((1 - mx) * T_HALF + c * C, C), :] = (
                recv_x[pl.ds(c * C, C), :].astype(jnp.float32))

        for c in range(K):
            rdma_y[c].wait_send()
            rdma_x[c].wait_send()

    return pl.pallas_call(
        body,
        out_shape=jax.ShapeDtypeStruct((T, D), jnp.float32),
        in_specs=[
            pl.BlockSpec(memory_space=pltpu.SMEM),
            pl.BlockSpec(memory_space=pltpu.VMEM),
            pl.BlockSpec(memory_space=pltpu.HBM),
        ],
        out_specs=pl.BlockSpec(memory_space=pltpu.VMEM),
        scratch_shapes=[
            pltpu.VMEM((T_HALF, D), jnp.float32),
            pltpu.VMEM((T_HALF, D), jnp.bfloat16),
            pltpu.VMEM((T_HALF, D), jnp.bfloat16),
            pltpu.VMEM((T_HALF, D), jnp.bfloat16),
            pltpu.VMEM((T_HALF, D), jnp.bfloat16),
            pltpu.SemaphoreType.DMA((2,)),
            pltpu.SemaphoreType.DMA((K,)),
            pltpu.SemaphoreType.DMA((K,)),
            pltpu.SemaphoreType.DMA((K,)),
            pltpu.SemaphoreType.DMA((K,)),
        ],
        compiler_params=pltpu.CompilerParams(collective_id=0),
    )(rows, okf, E)
